# baseline (speedup 1.0000x reference)
import functools

import jax
import jax.numpy as jnp
from jax import lax
from jax.experimental import pallas as pl
from jax.experimental.pallas import tpu as pltpu
from jax.experimental.pallas import tpu_sc as plsc

E = 8
D = 1024
F = 2048
T = 2048

BM = 256
NB = 24
NPAD = NB * BM
FC = 512
J = F // FC

NW = 32
DP = 128
DSUB = 32
CT = 64
CSUB = 32


def _router_body(x_ref, gw_ref, pos0_ref, pos1_ref, ws_ref,
                 be_ref, nb_ref):
    x = x_ref[...]
    logits = lax.dot_general(x, gw_ref[...], (((1,), (1,)), ((), ())),
                             preferred_element_type=jnp.float32)
    m = jnp.max(logits, axis=-1, keepdims=True)
    ex = jnp.exp(logits - m)
    probs = ex / jnp.sum(ex, axis=-1, keepdims=True)

    iota_e = lax.broadcasted_iota(jnp.int32, (T, E), 1)
    p0 = jnp.max(probs, axis=-1, keepdims=True)
    i0 = jnp.min(jnp.where(probs == p0, iota_e, E), axis=-1, keepdims=True)
    masked = jnp.where(iota_e == i0, -1.0, probs)
    p1 = jnp.max(masked, axis=-1, keepdims=True)
    i1 = jnp.min(jnp.where(masked == p1, iota_e, E), axis=-1, keepdims=True)
    s = p0 + p1
    w0 = p0 / s
    w1 = p1 / s

    oh0 = (iota_e == i0).astype(jnp.float32)
    oh1 = (iota_e == i1).astype(jnp.float32)

    rr = lax.broadcasted_iota(jnp.int32, (T, T), 0)
    cc = lax.broadcasted_iota(jnp.int32, (T, T), 1)
    ltri = (cc < rr).astype(jnp.float32)
    s0 = lax.dot_general(ltri, oh0, (((1,), (0,)), ((), ())),
                         preferred_element_type=jnp.float32)
    s1 = lax.dot_general(ltri, oh1, (((1,), (0,)), ((), ())),
                         preferred_element_type=jnp.float32)

    c0_row = jnp.sum(oh0, axis=0, keepdims=True)
    c1_row = jnp.sum(oh1, axis=0, keepdims=True)
    c_row = c0_row + c1_row
    blocks_row = jnp.floor((c_row + (BM - 1)) * (1.0 / BM))

    ia = lax.broadcasted_iota(jnp.int32, (E, E), 0)
    ib = lax.broadcasted_iota(jnp.int32, (E, E), 1)
    u8 = (ia < ib).astype(jnp.float32)
    po_row = BM * lax.dot_general(blocks_row, u8, (((1,), (0,)), ((), ())),
                                  preferred_element_type=jnp.float32)

    r0 = jnp.sum(s0 * oh0, axis=1, keepdims=True)
    r1 = jnp.sum((s1 + c0_row) * oh1, axis=1, keepdims=True)
    pos0 = jnp.sum(po_row * oh0, axis=1, keepdims=True) + r0
    pos1 = jnp.sum(po_row * oh1, axis=1, keepdims=True) + r1
    pos0i = pos0.astype(jnp.int32)
    pos1i = pos1.astype(jnp.int32)
    pos0_ref[...] = pos0i
    pos1_ref[...] = pos1i

    wchunks = []
    CW = 1024
    for cw in range(NPAD // CW):
        iota_c = lax.broadcasted_iota(jnp.int32, (T, CW), 1) + (cw * CW)
        m0 = (pos0i == iota_c).astype(jnp.float32)
        m1 = (pos1i == iota_c).astype(jnp.float32)
        wc = lax.dot_general(m0, w0, (((0,), (0,)), ((), ())),
                             preferred_element_type=jnp.float32)
        wc = wc + lax.dot_general(m1, w1, (((0,), (0,)), ((), ())),
                                  preferred_element_type=jnp.float32)
        wchunks.append(wc)
    ws_ref[...] = jnp.concatenate(wchunks, axis=0)

    ones_col = jnp.ones((T, 1), jnp.float32)
    c_col = lax.dot_general(oh0 + oh1, ones_col, (((0,), (0,)), ((), ())),
                            preferred_element_type=jnp.float32)
    blocks_col = jnp.floor((c_col + (BM - 1)) * (1.0 / BM))
    l8 = (ib < ia).astype(jnp.float32)
    cb_col = lax.dot_general(l8, blocks_col, (((1,), (0,)), ((), ())),
                             preferred_element_type=jnp.float32)
    fin_col = cb_col + blocks_col

    iota_b = lax.broadcasted_iota(jnp.int32, (E, NB), 1).astype(jnp.float32)
    raw = jnp.sum((iota_b >= fin_col).astype(jnp.float32),
                  axis=0, keepdims=True)
    iota8_row = lax.broadcasted_iota(jnp.int32, (1, E), 1).astype(jnp.float32)
    la = jnp.max(jnp.where(blocks_row > 0, iota8_row, 0.0),
                 axis=1, keepdims=True)
    be = jnp.where(raw >= E, la, raw)
    be_ref[...] = be.astype(jnp.int32)
    nb_ref[...] = jnp.sum(blocks_row, axis=1, keepdims=True).astype(jnp.int32)


def _dispatch_body(hid_hbm, posall_hbm, xs_hbm, xb_v, idx_v, sem):
    wid = lax.axis_index("s") * 2 + lax.axis_index("c")
    kid = wid // 16
    for c in range(DP // DSUB):
        pbase = wid * DP + c * DSUB
        tbase = pbase - kid * T
        pltpu.sync_copy(posall_hbm.at[pl.ds(pbase, DSUB)], idx_v)
        pltpu.sync_copy(hid_hbm.at[pl.ds(tbase, DSUB)], xb_v)
        pltpu.async_copy(xb_v, xs_hbm.at[idx_v], sem).wait()


def _expert_body(be_ref, nb_ref, x_ref, w1g_ref, w1u_ref, w2_ref, ws_ref,
                 y_ref):
    b = pl.program_id(0)
    j = pl.program_id(1)

    @pl.when(b < nb_ref[0])
    def _active():
        x = x_ref[...]
        g = lax.dot_general(x, w1g_ref[0], (((1,), (1,)), ((), ())),
                            preferred_element_type=jnp.float32)
        u = lax.dot_general(x, w1u_ref[0], (((1,), (1,)), ((), ())),
                            preferred_element_type=jnp.float32)
        act = g * jax.nn.sigmoid(g) * u
        y = lax.dot_general(act, w2_ref[0], (((1,), (1,)), ((), ())),
                            preferred_element_type=jnp.float32)

        @pl.when(j == 0)
        def _init():
            y_ref[...] = y

        @pl.when(jnp.logical_and(j > 0, j < J - 1))
        def _acc():
            y_ref[...] += y

        @pl.when(j == J - 1)
        def _fin():
            y_ref[...] = (y_ref[...] + y) * ws_ref[...]


def _combine_body(y_hbm, pos0_hbm, pos1_hbm, out_hbm,
                  i0_v, i1_v, y0_v, y1_v, sem):
    wid = lax.axis_index("s") * 2 + lax.axis_index("c")
    for c in range(CT // CSUB):
        base = wid * CT + c * CSUB
        pltpu.sync_copy(pos0_hbm.at[pl.ds(base, CSUB)], i0_v)
        pltpu.sync_copy(pos1_hbm.at[pl.ds(base, CSUB)], i1_v)
        pltpu.async_copy(y_hbm.at[i0_v], y0_v, sem).wait()
        pltpu.async_copy(y_hbm.at[i1_v], y1_v, sem).wait()

        def _row(r, _):
            def _col(q, _):
                a = y0_v[r, pl.ds(q * 16, 16)]
                bb = y1_v[r, pl.ds(q * 16, 16)]
                y0_v[r, pl.ds(q * 16, 16)] = a + bb
                return 0

            lax.fori_loop(0, D // 16, _col, 0)
            return 0

        lax.fori_loop(0, CSUB, _row, 0)
        pltpu.sync_copy(y0_v, out_hbm.at[pl.ds(base, CSUB)])


@jax.jit
def kernel(hidden_states, gate_w, w1, w2):
    pos0c, pos1c, ws, be, nb = pl.pallas_call(
        _router_body,
        out_shape=(
            jax.ShapeDtypeStruct((T, 1), jnp.int32),
            jax.ShapeDtypeStruct((T, 1), jnp.int32),
            jax.ShapeDtypeStruct((NPAD, 1), jnp.float32),
            jax.ShapeDtypeStruct((1, NB), jnp.int32),
            jax.ShapeDtypeStruct((1, 1), jnp.int32),
        ),
    )(hidden_states, gate_w)

    pos0 = pos0c.reshape(T)
    pos1 = pos1c.reshape(T)
    posall = jnp.concatenate([pos0, pos1], axis=0)

    mesh = plsc.VectorSubcoreMesh(core_axis_name="c", subcore_axis_name="s")
    dispatch = functools.partial(
        pl.kernel, mesh=mesh,
        out_type=jax.ShapeDtypeStruct((NPAD, D), jnp.float32),
        scratch_types=[
            pltpu.VMEM((DSUB, D), jnp.float32),
            pltpu.VMEM((DSUB,), jnp.int32),
            pltpu.SemaphoreType.DMA,
        ],
    )(_dispatch_body)
    xs = dispatch(hidden_states, posall)

    grid_spec = pltpu.PrefetchScalarGridSpec(
        num_scalar_prefetch=2,
        grid=(NB, J),
        in_specs=[
            pl.BlockSpec((BM, D), lambda b, j, be, nb: (b, 0)),
            pl.BlockSpec((1, FC, D), lambda b, j, be, nb: (be[b], j, 0)),
            pl.BlockSpec((1, FC, D), lambda b, j, be, nb: (be[b], J + j, 0)),
            pl.BlockSpec((1, D, FC), lambda b, j, be, nb: (be[b], 0, j)),
            pl.BlockSpec((BM, 1), lambda b, j, be, nb: (b, 0)),
        ],
        out_specs=pl.BlockSpec((BM, D), lambda b, j, be, nb: (b, 0)),
    )
    y_sorted = pl.pallas_call(
        _expert_body,
        grid_spec=grid_spec,
        out_shape=jax.ShapeDtypeStruct((NPAD, D), jnp.float32),
        compiler_params=pltpu.CompilerParams(
            dimension_semantics=("arbitrary", "arbitrary")),
    )(be.reshape(NB), nb.reshape(1), xs, w1, w1, w2, ws)

    combine = functools.partial(
        pl.kernel, mesh=mesh,
        out_type=jax.ShapeDtypeStruct((T, D), jnp.float32),
        scratch_types=[
            pltpu.VMEM((CSUB,), jnp.int32),
            pltpu.VMEM((CSUB,), jnp.int32),
            pltpu.VMEM((CSUB, D), jnp.float32),
            pltpu.VMEM((CSUB, D), jnp.float32),
            pltpu.SemaphoreType.DMA,
        ],
    )(_combine_body)
    out = combine(y_sorted, pos0, pos1)
    return out
